# baseline (speedup 1.0000x reference)
import functools
import jax
import jax.numpy as jnp
from jax import lax
from jax.experimental import pallas as pl
from jax.experimental.pallas import tpu as pltpu
from jax.experimental.pallas import tpu_sc as plsc

_TEMP = 0.07
_K = 10
_N = 100000
_M = 1024
_D = 64
_TILE = 400
_NTILES = _N // _TILE
_NW = 32
_GROUPS = _TILE // 16


def _unpack2(w):
    lo = lax.bitcast_convert_type(w << 16, jnp.float32)
    hi = lax.bitcast_convert_type(w & jnp.int32(-65536), jnp.float32)
    return lo, hi


def _rsqrt16(x):
    i = lax.bitcast_convert_type(x, jnp.int32)
    y = lax.bitcast_convert_type(jnp.int32(0x5F3759DF) - (i >> 1), jnp.float32)
    for _ in range(3):
        y = y * (1.5 - 0.5 * x * y * y)
    return y


@functools.partial(
    pl.kernel,
    mesh=plsc.VectorSubcoreMesh(core_axis_name="c", subcore_axis_name="s"),
    out_type=jax.ShapeDtypeStruct((_N,), jnp.float32),
    compiler_params=pltpu.CompilerParams(
        needs_layout_passes=False, disable_bounds_checks=True
    ),
    scratch_types=[
        pltpu.VMEM((_M * _D // 2,), jnp.int32),
        pltpu.VMEM((_M,), jnp.float32),
        pltpu.VMEM((_TILE * _D,), jnp.float32),
        pltpu.VMEM((_TILE,), jnp.int32),
        pltpu.VMEM((_TILE * _K,), jnp.int32),
        pltpu.VMEM((_TILE,), jnp.float32),
    ],
)
def _sc_kernel(spT_hbm, rp_hbm, idx_hbm, r_hbm, out_hbm,
               spT_v, scale_v, rp_v, idx_v, r_v, out_v):
    wid = lax.axis_index("s") * 2 + lax.axis_index("c")
    pltpu.sync_copy(spT_hbm, spT_v)

    def col_body(c, carry):
        cb = c * 16
        ss = jnp.zeros((16,), jnp.float32)
        for d2 in range(_D // 2):
            w = spT_v[pl.ds(d2 * _M + cb, 16)]
            lo, hi = _unpack2(w)
            ss = ss + lo * lo + hi * hi
        scale_v[pl.ds(cb, 16)] = _rsqrt16(ss)
        return carry

    lax.fori_loop(0, _M // 16, col_body, 0)

    def tile_body(j, carry):
        t = wid + _NW * j
        base = t * _TILE
        pltpu.sync_copy(rp_hbm.at[pl.ds(base * _D, _TILE * _D)], rp_v)
        pltpu.sync_copy(idx_hbm.at[pl.ds(base, _TILE)], idx_v)
        pltpu.sync_copy(r_hbm.at[pl.ds(base * _K, _TILE * _K)], r_v)

        @plsc.parallel_loop(0, _GROUPS, 1, unroll=2)
        def group_body(g):
            pb = g * 16
            lanes = lax.iota(jnp.int32, 16)
            ip = idx_v[pl.ds(pb, 16)]
            p = pb + lanes
            p10 = p * _K
            p64 = p * _D
            idxs = [ip]
            for k in range(_K):
                rk = plsc.load_gather(r_v, [p10 + k])
                idxs.append(jnp.where(rk >= ip, rk + 1, rk))
            accs = [jnp.zeros((16,), jnp.float32) for _ in range(_K + 1)]
            ssq = jnp.zeros((16,), jnp.float32)
            rot = lanes
            for q in range(_D // 2):
                perm2 = p64 + rot * 2
                rv0 = plsc.load_gather(rp_v, [perm2])
                rv1 = plsc.load_gather(rp_v, [perm2 + 1])
                rotm = rot * _M
                ssq = ssq + rv0 * rv0 + rv1 * rv1
                for k in range(_K + 1):
                    w = plsc.load_gather(spT_v, [idxs[k] + rotm])
                    lo, hi = _unpack2(w)
                    accs[k] = accs[k] + rv0 * lo + rv1 * hi
                rot = (rot + 1) & (_D // 2 - 1)
            rs = _rsqrt16(ssq)
            lg = []
            for k in range(_K + 1):
                sc = plsc.load_gather(scale_v, [idxs[k]])
                lg.append(accs[k] * rs * sc / _TEMP)
            pos = lg[0]
            s = jnp.zeros((16,), jnp.float32)
            for k in range(1, _K + 1):
                s = s + jnp.exp(lg[k] - pos)
            out_v[pl.ds(pb, 16)] = s

        pltpu.sync_copy(out_v, out_hbm.at[pl.ds(base, _TILE)])
        return carry

    nt = jnp.where(wid < _NTILES - (_NTILES // _NW) * _NW, _NTILES // _NW + 1,
                   _NTILES // _NW)
    lax.fori_loop(0, nt, tile_body, 0)


def _tc_reduce(s):
    x = s.reshape(100, 1000)

    def body(x_ref, o_ref):
        v = x_ref[...]
        tot = jnp.sum(jnp.log(1.0 + v)) * jnp.float32(1.0 / _N)
        o_ref[...] = tot.reshape(1, 1)

    return pl.pallas_call(
        body,
        out_shape=jax.ShapeDtypeStruct((1, 1), jnp.float32),
    )(x)


def kernel(superPoint_feat, rawPoint_feat, raw_to_super_index):
    n = rawPoint_feat.shape[0]
    m = superPoint_feat.shape[0]
    r = jax.random.randint(jax.random.key(42), (n, _K), 0, m - 1)
    spb = superPoint_feat.T.astype(jnp.bfloat16)
    lo = lax.bitcast_convert_type(spb[0::2], jnp.uint16).astype(jnp.uint32)
    hi = lax.bitcast_convert_type(spb[1::2], jnp.uint16).astype(jnp.uint32)
    spT = lax.bitcast_convert_type(lo | (hi << 16), jnp.int32).reshape(-1)
    rp = rawPoint_feat.reshape(-1)
    idx = raw_to_super_index.astype(jnp.int32)
    rf = r.astype(jnp.int32).reshape(-1)
    s = _sc_kernel(spT, rp, idx, rf)
    return _tc_reduce(s)[0, 0]


# speedup vs baseline: 1.5470x; 1.5470x over previous
import functools
import jax
import jax.numpy as jnp
from jax import lax
from jax.experimental import pallas as pl
from jax.experimental.pallas import tpu as pltpu
from jax.experimental.pallas import tpu_sc as plsc

_TEMP = 0.07
_K = 10
_N = 100000
_M = 1024
_D = 64
_TILE = 400
_NTILES = _N // _TILE
_NW = 32
_GROUPS = _TILE // 16


def _rsqrt16(x):
    i = lax.bitcast_convert_type(x, jnp.int32)
    y = lax.bitcast_convert_type(jnp.int32(0x5F3759DF) - (i >> 1), jnp.float32)
    for _ in range(3):
        y = y * (1.5 - 0.5 * x * y * y)
    return y


@functools.partial(
    pl.kernel,
    mesh=plsc.VectorSubcoreMesh(core_axis_name="c", subcore_axis_name="s"),
    out_type=jax.ShapeDtypeStruct((_N,), jnp.float32),
    compiler_params=pltpu.CompilerParams(
        needs_layout_passes=False, disable_bounds_checks=True
    ),
    scratch_types=[
        pltpu.VMEM((_M * _D,), jnp.float32),
        pltpu.VMEM((_M,), jnp.float32),
        pltpu.VMEM((_TILE * _D,), jnp.float32),
        pltpu.VMEM((_TILE,), jnp.int32),
        pltpu.VMEM((_TILE * _K,), jnp.int32),
        pltpu.VMEM((_TILE * _D,), jnp.float32),
        pltpu.VMEM((_TILE,), jnp.int32),
        pltpu.VMEM((_TILE * _K,), jnp.int32),
        pltpu.VMEM((_TILE,), jnp.float32),
        pltpu.SemaphoreType.DMA,
        pltpu.SemaphoreType.DMA,
    ],
)
def _sc_kernel(spT_hbm, rp_hbm, idx_hbm, r_hbm, out_hbm,
               spT_v, scale_v, rp_v, idx_v, r_v, rp_v2, idx_v2, r_v2, out_v,
               sem_a, sem_b):
    wid = lax.axis_index("s") * 2 + lax.axis_index("c")
    pltpu.sync_copy(spT_hbm, spT_v)

    def col_body(c, carry):
        cb = c * 16
        ss = jnp.zeros((16,), jnp.float32)
        for d in range(_D):
            v = spT_v[pl.ds(d * _M + cb, 16)]
            ss = ss + v * v
        scale_v[pl.ds(cb, 16)] = _rsqrt16(ss)
        return carry

    lax.fori_loop(0, _M // 16, col_body, 0)

    def dma_start(j, rp_d, idx_d, r_d, sem):
        t = wid + _NW * j
        base = t * _TILE
        pltpu.async_copy(rp_hbm.at[pl.ds(base * _D, _TILE * _D)], rp_d, sem)
        pltpu.async_copy(idx_hbm.at[pl.ds(base, _TILE)], idx_d, sem)
        pltpu.async_copy(r_hbm.at[pl.ds(base * _K, _TILE * _K)], r_d, sem)

    def dma_wait(rp_d, idx_d, r_d, sem):
        pltpu.make_async_copy(rp_hbm.at[pl.ds(0, _TILE * _D)], rp_d, sem).wait()
        pltpu.make_async_copy(idx_hbm.at[pl.ds(0, _TILE)], idx_d, sem).wait()
        pltpu.make_async_copy(r_hbm.at[pl.ds(0, _TILE * _K)], r_d, sem).wait()

    def compute_tile(j, rp_d, idx_d, r_d):
        t = wid + _NW * j
        base = t * _TILE

        def group_body(g, gcarry):
            pb = g * 16
            lanes = lax.iota(jnp.int32, 16)
            ip = idx_d[pl.ds(pb, 16)]
            p = pb + lanes
            p10 = p * _K
            p64 = p * _D
            idxs = [ip]
            for k in range(_K):
                rk = plsc.load_gather(r_d, [p10 + k])
                idxs.append(jnp.where(rk >= ip, rk + 1, rk))
            accs = [jnp.zeros((16,), jnp.float32) for _ in range(_K + 1)]
            ssq = jnp.zeros((16,), jnp.float32)
            rot = lanes
            for dd in range(_D):
                rv = plsc.load_gather(rp_d, [p64 + rot])
                rotm = rot * _M
                ssq = ssq + rv * rv
                for k in range(_K + 1):
                    sv = plsc.load_gather(spT_v, [idxs[k] + rotm])
                    accs[k] = accs[k] + rv * sv
                rot = (rot + 1) & (_D - 1)
            rs = _rsqrt16(ssq)
            lg = []
            for k in range(_K + 1):
                sc = plsc.load_gather(scale_v, [idxs[k]])
                lg.append(accs[k] * rs * sc / _TEMP)
            pos = lg[0]
            s = jnp.zeros((16,), jnp.float32)
            for k in range(1, _K + 1):
                s = s + jnp.exp(lg[k] - pos)
            out_v[pl.ds(pb, 16)] = s
            return gcarry

        lax.fori_loop(0, _GROUPS, group_body, 0)
        pltpu.sync_copy(out_v, out_hbm.at[pl.ds(base, _TILE)])

    nt = jnp.where(wid < _NTILES - (_NTILES // _NW) * _NW, _NTILES // _NW + 1,
                   _NTILES // _NW)
    dma_start(0, rp_v, idx_v, r_v, sem_a)

    def pair_body(jj, carry):
        j0 = 2 * jj
        j1 = j0 + 1

        @pl.when(j0 < nt)
        def _():
            dma_wait(rp_v, idx_v, r_v, sem_a)

            @pl.when(j1 < nt)
            def _():
                dma_start(j1, rp_v2, idx_v2, r_v2, sem_b)

            compute_tile(j0, rp_v, idx_v, r_v)

            @pl.when(j0 + 2 < nt)
            def _():
                dma_start(j0 + 2, rp_v, idx_v, r_v, sem_a)

        @pl.when(j1 < nt)
        def _():
            dma_wait(rp_v2, idx_v2, r_v2, sem_b)
            compute_tile(j1, rp_v2, idx_v2, r_v2)

            @pl.when(j1 + 2 < nt)
            def _():
                dma_start(j1 + 2, rp_v2, idx_v2, r_v2, sem_b)

        return carry

    lax.fori_loop(0, (_NTILES // _NW + 2) // 2, pair_body, 0)


def _tc_reduce(s):
    x = s.reshape(100, 1000)

    def body(x_ref, o_ref):
        v = x_ref[...]
        tot = jnp.sum(jnp.log(1.0 + v)) * jnp.float32(1.0 / _N)
        o_ref[...] = tot.reshape(1, 1)

    return pl.pallas_call(
        body,
        out_shape=jax.ShapeDtypeStruct((1, 1), jnp.float32),
    )(x)


def kernel(superPoint_feat, rawPoint_feat, raw_to_super_index):
    n = rawPoint_feat.shape[0]
    m = superPoint_feat.shape[0]
    r = jax.random.randint(jax.random.key(42), (n, _K), 0, m - 1)
    spT = superPoint_feat.T.reshape(-1)
    rp = rawPoint_feat.reshape(-1)
    idx = raw_to_super_index.astype(jnp.int32)
    rf = r.astype(jnp.int32).reshape(-1)
    s = _sc_kernel(spT, rp, idx, rf)
    return _tc_reduce(s)[0, 0]
